# baseline (speedup 1.0000x reference)
import jax
import jax.numpy as jnp
from jax import lax
from jax.experimental import pallas as pl
from jax.experimental.pallas import tpu as pltpu
from jax.experimental.pallas import tpu_sc as plsc

V_DIM = 100000
EMB_DIM = 128
BATCH = 16384
CTX = 50

NC = 2
NS = 16
NW = NC * NS
LANES = 16

ROWS_PER_W = BATCH // NW
ROWS_PER_STEP = 1
IDX_PER_STEP = ROWS_PER_STEP * CTX
STEPS = ROWS_PER_W // ROWS_PER_STEP
NJ = EMB_DIM // LANES // 2
UNROLL = 7
NBUF = 8


def _cbow_body(x_hbm, table_hbm, out_hbm, idx_all, rows_v, out_v, gsems, osems):
    wid = lax.axis_index("s") * NC + lax.axis_index("c")
    obase = wid * ROWS_PER_W

    pltpu.sync_copy(x_hbm.at[pl.ds(obase, ROWS_PER_W)], idx_all)

    def gather(step, buf):
        return pltpu.async_copy(
            table_hbm.at[idx_all.at[step]], rows_v.at[buf], gsems[buf])

    for k in range(NBUF):
        gather(k, k)

    def outer(g6, carry):
        for b in range(NBUF):
            g = NBUF * g6 + b
            pltpu.make_async_copy(
                table_hbm.at[idx_all.at[g]], rows_v.at[b], gsems[b]).wait()

            @pl.when(g >= NBUF)
            def _():
                pltpu.make_async_copy(
                    out_v.at[b], out_hbm.at[pl.ds(obase, ROWS_PER_STEP)],
                    osems[b]).wait()

            for r in range(ROWS_PER_STEP):
                base = CTX * r
                accs = tuple(
                    rows_v[b, base, pl.ds(LANES * j, LANES)] for j in range(NJ))

                def inner(c, accs):
                    row = base + 1 + c * UNROLL
                    for u in range(UNROLL):
                        accs = tuple(
                            accs[j] + rows_v[b, row + u, pl.ds(LANES * j, LANES)]
                            for j in range(NJ))
                    return accs

                accs = lax.fori_loop(0, (CTX - 1) // UNROLL, inner, accs)
                for j in range(NJ):
                    out_v[b, r, pl.ds(LANES * j, LANES)] = (
                        accs[j] * (1.0 / CTX))

            pltpu.async_copy(
                out_v.at[b],
                out_hbm.at[pl.ds(obase + g * ROWS_PER_STEP, ROWS_PER_STEP)],
                osems[b])

            @pl.when(g + NBUF < STEPS)
            def _():
                gather(g + NBUF, b)
        return carry

    lax.fori_loop(0, STEPS // NBUF, outer, 0)

    for g in range(STEPS - STEPS % NBUF, STEPS):
        b = g % NBUF
        pltpu.make_async_copy(
            table_hbm.at[idx_all.at[g]], rows_v.at[b], gsems[b]).wait()
        pltpu.make_async_copy(
            out_v.at[b], out_hbm.at[pl.ds(obase, ROWS_PER_STEP)],
            osems[b]).wait()
        for r in range(ROWS_PER_STEP):
            base = CTX * r
            accs = tuple(
                rows_v[b, base, pl.ds(LANES * j, LANES)] for j in range(NJ))

            def inner(c, accs):
                row = base + 1 + c * UNROLL
                for u in range(UNROLL):
                    accs = tuple(
                        accs[j] + rows_v[b, row + u, pl.ds(LANES * j, LANES)]
                        for j in range(NJ))
                return accs

            accs = lax.fori_loop(0, (CTX - 1) // UNROLL, inner, accs)
            for j in range(NJ):
                out_v[b, r, pl.ds(LANES * j, LANES)] = accs[j] * (1.0 / CTX)
        pltpu.async_copy(
            out_v.at[b],
            out_hbm.at[pl.ds(obase + g * ROWS_PER_STEP, ROWS_PER_STEP)],
            osems[b])

    for b in range(NBUF):
        pltpu.make_async_copy(
            out_v.at[b], out_hbm.at[pl.ds(obase, ROWS_PER_STEP)],
            osems[b]).wait()


@jax.jit
def kernel(x, table):
    mesh = plsc.VectorSubcoreMesh(core_axis_name="c", subcore_axis_name="s",
                                  num_cores=NC, num_subcores=NS)
    f = pl.kernel(
        _cbow_body,
        out_type=jax.ShapeDtypeStruct((BATCH, EMB_DIM), jnp.float32),
        mesh=mesh,
        scratch_types=[
            pltpu.VMEM((STEPS, IDX_PER_STEP), jnp.int32),
            pltpu.VMEM((NBUF, IDX_PER_STEP, EMB_DIM), jnp.float32),
            pltpu.VMEM((NBUF, ROWS_PER_STEP, EMB_DIM), jnp.float32),
            [pltpu.SemaphoreType.DMA] * NBUF,
            [pltpu.SemaphoreType.DMA] * NBUF,
        ],
    )
    if x.dtype != jnp.int32:
        x = x.astype(jnp.int32)
    return f(x, table)


# speedup vs baseline: 1.0519x; 1.0519x over previous
import jax
import jax.numpy as jnp
from jax import lax
from jax.experimental import pallas as pl
from jax.experimental.pallas import tpu as pltpu
from jax.experimental.pallas import tpu_sc as plsc

V_DIM = 100000
EMB_DIM = 128
BATCH = 16384
CTX = 50

NC = 2
NS = 16
NW = NC * NS
LANES = 16

STEPS = BATCH // NW
NJ = EMB_DIM // LANES
UNROLL = 7
NBUF = 8

assert STEPS % NBUF == 0 and (CTX - 1) % UNROLL == 0


def _cbow_body(x_hbm, table_hbm, out_hbm, idx_all, rows_v, out_v, gsems, osems):
    wid = lax.axis_index("s") * NC + lax.axis_index("c")
    obase = wid * STEPS

    pltpu.sync_copy(x_hbm.at[pl.ds(obase, STEPS)], idx_all)

    def gather(step, buf):
        return pltpu.async_copy(
            table_hbm.at[idx_all.at[step]], rows_v.at[buf], gsems[buf])

    for k in range(NBUF):
        gather(k, k)

    def outer(gg, carry):
        for b in range(NBUF):
            g = NBUF * gg + b
            pltpu.make_async_copy(
                table_hbm.at[idx_all.at[g]], rows_v.at[b], gsems[b]).wait()

            @pl.when(g >= NBUF)
            def _():
                pltpu.make_async_copy(
                    out_v.at[b], out_hbm.at[pl.ds(obase, 1)], osems[b]).wait()

            accs = tuple(
                rows_v[b, 0, pl.ds(LANES * j, LANES)] for j in range(NJ))

            def inner(c, accs):
                row = 1 + c * UNROLL
                for u in range(UNROLL):
                    accs = tuple(
                        accs[j] + rows_v[b, row + u, pl.ds(LANES * j, LANES)]
                        for j in range(NJ))
                return accs

            accs = lax.fori_loop(0, (CTX - 1) // UNROLL, inner, accs)
            for j in range(NJ):
                out_v[b, 0, pl.ds(LANES * j, LANES)] = accs[j] * (1.0 / CTX)

            pltpu.async_copy(
                out_v.at[b], out_hbm.at[pl.ds(obase + g, 1)], osems[b])

            @pl.when(g + NBUF < STEPS)
            def _():
                gather(g + NBUF, b)
        return carry

    lax.fori_loop(0, STEPS // NBUF, outer, 0)

    for b in range(NBUF):
        pltpu.make_async_copy(
            out_v.at[b], out_hbm.at[pl.ds(obase, 1)], osems[b]).wait()


@jax.jit
def kernel(x, table):
    mesh = plsc.VectorSubcoreMesh(core_axis_name="c", subcore_axis_name="s",
                                  num_cores=NC, num_subcores=NS)
    f = pl.kernel(
        _cbow_body,
        out_type=jax.ShapeDtypeStruct((BATCH, EMB_DIM), jnp.float32),
        mesh=mesh,
        scratch_types=[
            pltpu.VMEM((STEPS, CTX), jnp.int32),
            pltpu.VMEM((NBUF, CTX, EMB_DIM), jnp.float32),
            pltpu.VMEM((NBUF, 1, EMB_DIM), jnp.float32),
            [pltpu.SemaphoreType.DMA] * NBUF,
            [pltpu.SemaphoreType.DMA] * NBUF,
        ],
    )
    if x.dtype != jnp.int32:
        x = x.astype(jnp.int32)
    return f(x, table)
